# baseline (speedup 1.0000x reference)
import functools

import jax
import jax.numpy as jnp
from jax import lax
from jax.experimental import pallas as pl
from jax.experimental.pallas import tpu as pltpu
from jax.experimental.pallas import tpu_sc as plsc

_NUM_CLASSES = 2
_NEG_POS_RATIO = 3
_THRESHOLD = 0.35
_V0, _V1 = 0.1, 0.2
_BISECT_ITERS = 30


def _smooth_l1(x):
    ax = jnp.abs(x)
    return jnp.where(ax < 1.0, 0.5 * ax * ax, ax - 0.5)


def _stage1_body(pred_ref, an_ref, tg_ref,
                 lossc_ref, npos_ref, pce_ref, bxl_ref, ldl_ref, nprow_ref):
    A = an_ref.shape[1]
    n = tg_ref.shape[1]
    predT = pred_ref[0]

    an = an_ref[...]
    px1, py1 = an[0:1, :], an[1:2, :]
    px2, py2 = an[2:3, :], an[3:4, :]

    tg = tg_ref[0]
    tx1, ty1 = tg[:, 0:1], tg[:, 1:2]
    tx2, ty2 = tg[:, 2:3], tg[:, 3:4]

    iw = jnp.clip(jnp.minimum(tx2, px2) - jnp.maximum(tx1, px1), 0.0, None)
    ih = jnp.clip(jnp.minimum(ty2, py2) - jnp.maximum(ty1, py1), 0.0, None)
    inter = iw * ih
    area_t = (tx2 - tx1) * (ty2 - ty1)
    ov = inter / (area_t + an[4:5, :] - inter)

    iota_n = lax.broadcasted_iota(jnp.int32, (n, A), 0)
    iota_a = lax.broadcasted_iota(jnp.int32, (n, A), 1)

    ovb = lax.bitcast_convert_type(ov, jnp.int32)
    kp = jnp.bitwise_or(jnp.bitwise_and(ovb, -32), (n - 1) - iota_n)
    mx = jnp.max(kp, axis=0, keepdims=True)
    bti = (n - 1) - jnp.bitwise_and(mx, n - 1)
    bto = lax.bitcast_convert_type(jnp.bitwise_and(mx, -32), jnp.float32)

    rmax = jnp.max(ov, axis=1, keepdims=True)
    bpi = jnp.min(jnp.where(ov == rmax, iota_a, A), axis=1, keepdims=True)

    eqf = bpi == iota_a
    fi = jnp.max(jnp.where(eqf, iota_n, -1), axis=0, keepdims=True)
    forced = fi >= 0
    bti = jnp.where(forced, fi, bti)

    pos = jnp.logical_or(forced, bto >= _THRESHOLD)
    posf = pos.astype(jnp.float32)

    oh = (bti == iota_n).astype(jnp.float32)
    table = tg[:, 0:14]
    matched = lax.dot_general(table, oh, (((0,), (0,)), ((), ())),
                              preferred_element_type=jnp.float32)

    gcxy = ((matched[0:2, :] + matched[2:4, :]) * 0.5
            - an[5:7, :]) * an[7:9, :]
    gwh = jnp.log(jnp.maximum(
        (matched[2:4, :] - matched[0:2, :]) * an[9:11, :], 1e-8)) * (1.0 / _V1)
    loc = jnp.concatenate([gcxy, gwh], axis=0)
    bxl_ref[0, 0, 0] = jnp.sum(_smooth_l1(predT[2:6, :] - loc) * posf)

    gld = (matched[4:14, :] - an[11:21, :]) * an[21:31, :]
    ldl_ref[0, 0, 0] = jnp.sum(
        _smooth_l1(predT[6:16, :] - gld) * posf)

    l0 = predT[0:1, :]
    l1 = predT[1:2, :]
    m = jnp.maximum(l0, l1)
    logz = m + jnp.log1p(jnp.exp(-jnp.abs(l0 - l1)))

    npb = jnp.sum(posf)
    npos_ref[0, 0, 0] = npb
    nprow_ref[...] = jnp.full((1, 1, 16), npb, jnp.float32)
    pce_ref[0, 0, 0] = jnp.sum((logz - l1) * posf)
    lossc_ref[...] = ((logz - l0) * (1.0 - posf))[None]


_L = 16
_SC_ITERS = 24
_UNROLL = 10


def _bfly_max(x, iota):
    for d in (1, 2, 4, 8):
        x = jnp.maximum(x, x.at[jnp.bitwise_xor(iota, d)].get(
            mode="promise_in_bounds"))
    return x


def _bfly_sum(x, iota):
    for d in (1, 2, 4, 8):
        x = x + x.at[jnp.bitwise_xor(iota, d)].get(mode="promise_in_bounds")
    return x


def _sc_mine_body(lossc_hbm, nprow_hbm, out_hbm, row_v, np_v, tmp_v):
    B, A = lossc_hbm.shape
    outer = A // _L // _UNROLL
    wid = lax.axis_index("s") * 2 + lax.axis_index("c")

    @pl.when(wid < B)
    def _():
        iota = lax.iota(jnp.int32, _L)
        zeros = jnp.zeros((_L,), jnp.float32)

        pltpu.sync_copy(lossc_hbm.at[wid], row_v)
        pltpu.sync_copy(nprow_hbm.at[wid], np_v)
        k = jnp.minimum(np_v[...] * float(_NEG_POS_RATIO), float(A - 1))

        def mx_body(i, m):
            for u in range(_UNROLL):
                m = jnp.maximum(m, row_v[pl.ds((i * _UNROLL + u) * _L, _L)])
            return m
        hi = _bfly_max(lax.fori_loop(0, outer, mx_body, zeros), iota) + 1.0

        def it(_, carry):
            lo, hi = carry
            mid = (lo + hi) * 0.5

            def cnt_body(i, acc):
                for u in range(_UNROLL):
                    v = row_v[pl.ds((i * _UNROLL + u) * _L, _L)]
                    acc = acc + jnp.where(v > mid, 1.0, 0.0)
                return acc
            cnt = _bfly_sum(lax.fori_loop(0, outer, cnt_body, zeros), iota)
            ge = cnt >= k
            return jnp.where(ge, mid, lo), jnp.where(ge, hi, mid)
        lo, _ = lax.fori_loop(0, _SC_ITERS, it, (zeros, hi))

        def fin_body(i, carry):
            accc, accs = carry
            for u in range(_UNROLL):
                v = row_v[pl.ds((i * _UNROLL + u) * _L, _L)]
                sel = v > lo
                accc = accc + jnp.where(sel, 1.0, 0.0)
                accs = accs + jnp.where(sel, v, 0.0)
            return accc, accs
        accc, accs = lax.fori_loop(0, outer, fin_body, (zeros, zeros))
        cnt = _bfly_sum(accc, iota)
        s0 = _bfly_sum(accs, iota)
        tmp_v[...] = s0 + (k - cnt) * lo
        pltpu.sync_copy(tmp_v, out_hbm.at[wid])


@jax.jit
def kernel(pred_logits, pred_boxes, pred_landmarks, anchor_boxes, targets):
    B, A, _ = pred_logits.shape
    n = targets.shape[1]
    pred_all = jnp.transpose(
        jnp.concatenate([pred_logits, pred_boxes, pred_landmarks], axis=-1),
        (0, 2, 1))

    pcx, pcy = anchor_boxes[:, 0], anchor_boxes[:, 1]
    pw, ph = anchor_boxes[:, 2], anchor_boxes[:, 3]
    rvw, rvh = 1.0 / (_V0 * pw), 1.0 / (_V0 * ph)
    px1, py1 = pcx - pw * 0.5, pcy - ph * 0.5
    px2, py2 = pcx + pw * 0.5, pcy + ph * 0.5
    an_ext = jnp.stack(
        [px1, py1, px2, py2,
         (px2 - px1) * (py2 - py1), pcx, pcy, rvw, rvh, 1.0 / pw, 1.0 / ph]
        + [pcx, pcy] * 5 + [rvw, rvh] * 5, axis=0)

    smem11 = pl.BlockSpec((1, 1, 1), lambda b: (b, 0, 0),
                          memory_space=pltpu.SMEM)
    lossc, npos, pce, bxl, ldl, nprow = pl.pallas_call(
        _stage1_body,
        grid=(B,),
        in_specs=[
            pl.BlockSpec((1, 16, A), lambda b: (b, 0, 0)),
            pl.BlockSpec((31, A), lambda b: (0, 0)),
            pl.BlockSpec((1, n, 15), lambda b: (b, 0, 0)),
        ],
        out_specs=[
            pl.BlockSpec((1, 1, A), lambda b: (b, 0, 0)),
            smem11, smem11, smem11, smem11,
            pl.BlockSpec((1, 1, 16), lambda b: (b, 0, 0)),
        ],
        out_shape=[
            jax.ShapeDtypeStruct((B, 1, A), jnp.float32),
            jax.ShapeDtypeStruct((B, 1, 1), jnp.float32),
            jax.ShapeDtypeStruct((B, 1, 1), jnp.float32),
            jax.ShapeDtypeStruct((B, 1, 1), jnp.float32),
            jax.ShapeDtypeStruct((B, 1, 1), jnp.float32),
            jax.ShapeDtypeStruct((B, 1, 16), jnp.float32),
        ],
    )(pred_all, an_ext, targets)

    mine = pl.kernel(
        _sc_mine_body,
        mesh=plsc.VectorSubcoreMesh(core_axis_name="c", subcore_axis_name="s"),
        out_type=jax.ShapeDtypeStruct((B, _L), jnp.float32),
        scratch_types=[
            pltpu.VMEM((A,), jnp.float32),
            pltpu.VMEM((_L,), jnp.float32),
            pltpu.VMEM((_L,), jnp.float32),
        ],
    )
    negrows = mine(lossc.reshape(B, A), nprow.reshape(B, 16))

    npv = npos.reshape(B)
    n_tot = jnp.maximum(jnp.sum(npv), 1.0)
    cls = (jnp.sum(pce.reshape(B)) + jnp.sum(negrows[:, 0])) / n_tot
    box = jnp.sum(bxl.reshape(B)) / n_tot
    ldm = jnp.sum(ldl.reshape(B)) / n_tot
    return (cls, box, ldm)


# speedup vs baseline: 1.0588x; 1.0588x over previous
import functools

import jax
import jax.numpy as jnp
from jax import lax
from jax.experimental import pallas as pl
from jax.experimental.pallas import tpu as pltpu
from jax.experimental.pallas import tpu_sc as plsc

_NUM_CLASSES = 2
_NEG_POS_RATIO = 3
_THRESHOLD = 0.35
_V0, _V1 = 0.1, 0.2
_BISECT_ITERS = 30


def _smooth_l1(x):
    ax = jnp.abs(x)
    return jnp.where(ax < 1.0, 0.5 * ax * ax, ax - 0.5)


def _stage1_body(pred_ref, an_ref, tg_ref,
                 lossc_ref, npos_ref, pce_ref, bxl_ref, ldl_ref, nprow_ref):
    A = an_ref.shape[1]
    n = tg_ref.shape[1]
    predT = pred_ref[0]

    an = an_ref[...]
    px1, py1 = an[0:1, :], an[1:2, :]
    px2, py2 = an[2:3, :], an[3:4, :]

    tg = tg_ref[0]
    tx1, ty1 = tg[:, 0:1], tg[:, 1:2]
    tx2, ty2 = tg[:, 2:3], tg[:, 3:4]

    iw = jnp.clip(jnp.minimum(tx2, px2) - jnp.maximum(tx1, px1), 0.0, None)
    ih = jnp.clip(jnp.minimum(ty2, py2) - jnp.maximum(ty1, py1), 0.0, None)
    inter = iw * ih
    area_t = (tx2 - tx1) * (ty2 - ty1)
    ov = inter / (area_t + an[4:5, :] - inter)

    iota_n = lax.broadcasted_iota(jnp.int32, (n, A), 0)
    iota_a = lax.broadcasted_iota(jnp.int32, (n, A), 1)

    ovb = lax.bitcast_convert_type(ov, jnp.int32)
    kp = jnp.bitwise_or(jnp.bitwise_and(ovb, -32), (n - 1) - iota_n)
    mx = jnp.max(kp, axis=0, keepdims=True)
    bti = (n - 1) - jnp.bitwise_and(mx, n - 1)
    bto = lax.bitcast_convert_type(jnp.bitwise_and(mx, -32), jnp.float32)

    rmax = jnp.max(ov, axis=1, keepdims=True)
    bpi = jnp.min(jnp.where(ov == rmax, iota_a, A), axis=1, keepdims=True)

    eqf = bpi == iota_a
    fi = jnp.max(jnp.where(eqf, iota_n, -1), axis=0, keepdims=True)
    forced = fi >= 0
    bti = jnp.where(forced, fi, bti)

    pos = jnp.logical_or(forced, bto >= _THRESHOLD)
    posf = pos.astype(jnp.float32)

    oh = (bti == iota_n).astype(jnp.float32)
    table = tg[:, 0:14]
    matched = lax.dot_general(table, oh, (((0,), (0,)), ((), ())),
                              preferred_element_type=jnp.float32)

    gcxy = ((matched[0:2, :] + matched[2:4, :]) * 0.5
            - an[5:7, :]) * an[7:9, :]
    gwh = jnp.log(jnp.maximum(
        (matched[2:4, :] - matched[0:2, :]) * an[9:11, :], 1e-8)) * (1.0 / _V1)
    loc = jnp.concatenate([gcxy, gwh], axis=0)
    bxl_ref[0, 0, 0] = jnp.sum(_smooth_l1(predT[2:6, :] - loc) * posf)

    gld = (matched[4:14, :] - an[11:21, :]) * an[21:31, :]
    ldl_ref[0, 0, 0] = jnp.sum(
        _smooth_l1(predT[6:16, :] - gld) * posf)

    l0 = predT[0:1, :]
    l1 = predT[1:2, :]
    m = jnp.maximum(l0, l1)
    logz = m + jnp.log1p(jnp.exp(-jnp.abs(l0 - l1)))

    npb = jnp.sum(posf)
    npos_ref[0, 0, 0] = npb
    nprow_ref[...] = jnp.full((1, 1, 16), npb, jnp.float32)
    pce_ref[0, 0, 0] = jnp.sum((logz - l1) * posf)
    lossc_ref[...] = ((logz - l0) * (1.0 - posf))[None]


_L = 16
_SC_ITERS = 16
_UNROLL = 10


def _bfly_max(x, iota):
    for d in (1, 2, 4, 8):
        x = jnp.maximum(x, x.at[jnp.bitwise_xor(iota, d)].get(
            mode="promise_in_bounds"))
    return x


def _bfly_sum(x, iota):
    for d in (1, 2, 4, 8):
        x = x + x.at[jnp.bitwise_xor(iota, d)].get(mode="promise_in_bounds")
    return x


def _sc_mine_body(lossc_hbm, nprow_hbm, out_hbm, row_v, np_v, tmp_v):
    B, A = lossc_hbm.shape
    outer = A // _L // _UNROLL
    wid = lax.axis_index("s") * 2 + lax.axis_index("c")

    @pl.when(wid < B)
    def _():
        iota = lax.iota(jnp.int32, _L)
        zeros = jnp.zeros((_L,), jnp.float32)

        pltpu.sync_copy(lossc_hbm.at[wid], row_v)
        pltpu.sync_copy(nprow_hbm.at[wid], np_v)
        k = jnp.minimum(np_v[...] * float(_NEG_POS_RATIO), float(A - 1))

        def mx_body(i, m):
            for u in range(_UNROLL):
                m = jnp.maximum(m, row_v[pl.ds((i * _UNROLL + u) * _L, _L)])
            return m
        hi = _bfly_max(lax.fori_loop(0, outer, mx_body, zeros), iota) + 1.0

        def it(_, carry):
            lo, hi = carry
            mid = (lo + hi) * 0.5

            def cnt_body(i, acc):
                for u in range(_UNROLL):
                    v = row_v[pl.ds((i * _UNROLL + u) * _L, _L)]
                    acc = acc + jnp.where(v > mid, 1.0, 0.0)
                return acc
            cnt = _bfly_sum(lax.fori_loop(0, outer, cnt_body, zeros), iota)
            ge = cnt >= k
            return jnp.where(ge, mid, lo), jnp.where(ge, hi, mid)
        lo, _ = lax.fori_loop(0, _SC_ITERS, it, (zeros, hi))

        def fin_body(i, carry):
            accc, accs = carry
            for u in range(_UNROLL):
                v = row_v[pl.ds((i * _UNROLL + u) * _L, _L)]
                sel = v > lo
                accc = accc + jnp.where(sel, 1.0, 0.0)
                accs = accs + jnp.where(sel, v, 0.0)
            return accc, accs
        accc, accs = lax.fori_loop(0, outer, fin_body, (zeros, zeros))
        cnt = _bfly_sum(accc, iota)
        s0 = _bfly_sum(accs, iota)
        tmp_v[...] = s0 + (k - cnt) * lo
        pltpu.sync_copy(tmp_v, out_hbm.at[wid])


@jax.jit
def kernel(pred_logits, pred_boxes, pred_landmarks, anchor_boxes, targets):
    B, A, _ = pred_logits.shape
    n = targets.shape[1]
    pred_all = jnp.transpose(
        jnp.concatenate([pred_logits, pred_boxes, pred_landmarks], axis=-1),
        (0, 2, 1))

    pcx, pcy = anchor_boxes[:, 0], anchor_boxes[:, 1]
    pw, ph = anchor_boxes[:, 2], anchor_boxes[:, 3]
    rvw, rvh = 1.0 / (_V0 * pw), 1.0 / (_V0 * ph)
    px1, py1 = pcx - pw * 0.5, pcy - ph * 0.5
    px2, py2 = pcx + pw * 0.5, pcy + ph * 0.5
    an_ext = jnp.stack(
        [px1, py1, px2, py2,
         (px2 - px1) * (py2 - py1), pcx, pcy, rvw, rvh, 1.0 / pw, 1.0 / ph]
        + [pcx, pcy] * 5 + [rvw, rvh] * 5, axis=0)

    smem11 = pl.BlockSpec((1, 1, 1), lambda b: (b, 0, 0),
                          memory_space=pltpu.SMEM)
    lossc, npos, pce, bxl, ldl, nprow = pl.pallas_call(
        _stage1_body,
        grid=(B,),
        in_specs=[
            pl.BlockSpec((1, 16, A), lambda b: (b, 0, 0)),
            pl.BlockSpec((31, A), lambda b: (0, 0)),
            pl.BlockSpec((1, n, 15), lambda b: (b, 0, 0)),
        ],
        out_specs=[
            pl.BlockSpec((1, 1, A), lambda b: (b, 0, 0)),
            smem11, smem11, smem11, smem11,
            pl.BlockSpec((1, 1, 16), lambda b: (b, 0, 0)),
        ],
        out_shape=[
            jax.ShapeDtypeStruct((B, 1, A), jnp.float32),
            jax.ShapeDtypeStruct((B, 1, 1), jnp.float32),
            jax.ShapeDtypeStruct((B, 1, 1), jnp.float32),
            jax.ShapeDtypeStruct((B, 1, 1), jnp.float32),
            jax.ShapeDtypeStruct((B, 1, 1), jnp.float32),
            jax.ShapeDtypeStruct((B, 1, 16), jnp.float32),
        ],
    )(pred_all, an_ext, targets)

    mine = pl.kernel(
        _sc_mine_body,
        mesh=plsc.VectorSubcoreMesh(core_axis_name="c", subcore_axis_name="s"),
        out_type=jax.ShapeDtypeStruct((B, _L), jnp.float32),
        scratch_types=[
            pltpu.VMEM((A,), jnp.float32),
            pltpu.VMEM((_L,), jnp.float32),
            pltpu.VMEM((_L,), jnp.float32),
        ],
    )
    negrows = mine(lossc.reshape(B, A), nprow.reshape(B, 16))

    npv = npos.reshape(B)
    n_tot = jnp.maximum(jnp.sum(npv), 1.0)
    cls = (jnp.sum(pce.reshape(B)) + jnp.sum(negrows[:, 0])) / n_tot
    box = jnp.sum(bxl.reshape(B)) / n_tot
    ldm = jnp.sum(ldl.reshape(B)) / n_tot
    return (cls, box, ldm)
